# baseline (speedup 1.0000x reference)
import jax
import jax.numpy as jnp
from jax import lax
from jax.experimental import pallas as pl
from jax.experimental.pallas import tpu as pltpu
from jax.experimental.pallas import tpu_sc as plsc

N = 16384
D = 256
LAYERS = 3
NT = 16
LANES = 16
SLABS = 4
SLAB_D = D // SLABS
BATCH = 128
CH = 8
NGB = 4
NSB = 2
ROWS_PER_TILE = N // NT


def _layer_body(x_in, idx4, rows_t, vals_t, out_hbm,
                shared,
                im0, rm0, vm0, im1, rm1, vm1,
                gb0, gb1, gb2, gb3, sb0, sb1,
                msem0, msem1, gsem0, gsem1, gsem2, gsem3, ssem0, ssem1):
    nb = idx4.shape[2]
    nc = nb // CH
    c = lax.axis_index("c")
    w = lax.axis_index("s")
    mbufs = ((im0, rm0, vm0), (im1, rm1, vm1))
    msems = (msem0, msem1)
    gbufs = (gb0, gb1, gb2, gb3)
    gsems = (gsem0, gsem1, gsem2, gsem3)
    sbufs = (sb0, sb1)
    ssems = (ssem0, ssem1)

    zero16 = jnp.zeros((LANES,), jnp.float32)

    def meta_copies(s, ch, m):
        sl = pl.ds(ch * CH, CH)
        return (
            pltpu.make_async_copy(idx4.at[s, w, sl], mbufs[m][0], msems[m]),
            pltpu.make_async_copy(rows_t.at[w, sl], mbufs[m][1], msems[m]),
            pltpu.make_async_copy(vals_t.at[w, sl], mbufs[m][2], msems[m]),
        )

    def stage_meta(s, ch, m):
        for cp in meta_copies(s, ch, m):
            cp.start()

    def wait_meta(s, ch, m):
        for cp in meta_copies(s, ch, m):
            cp.wait()

    def visit(s, ch, k, m, has_next, guard_first):
        bi = ch * CH + k
        gslot = k % NGB
        sslot = k % NSB
        gb = gbufs[gslot]
        sb = sbufs[sslot]
        mn = 1 - m

        if k == NGB and has_next:
            wait_meta(s, ch + 1, mn)

        pltpu.make_async_copy(
            x_in.at[mbufs[m][0].at[k]], gb, gsems[gslot]).wait()

        def _wait_scatter():
            pltpu.make_async_copy(
                sb, shared.at[mbufs[m][1].at[k]], ssems[sslot]).wait()

        if guard_first and k < NSB:
            pl.when(bi >= NSB)(_wait_scatter)
        else:
            _wait_scatter()

        @pl.loop(0, BATCH // LANES)
        def _mul(g):
            v16 = mbufs[m][2][k, pl.ds(g * LANES, LANES)]

            @pl.loop(0, LANES, step=4)
            def _mul_j(j0):
                for dj in range(4):
                    j = j0 + dj
                    splat = v16.at[jnp.full((LANES,), 0, jnp.int32) + j].get(
                        mode="promise_in_bounds")
                    r = g * LANES + j
                    a = [gb[r, pl.ds(q * LANES, LANES)]
                         for q in range(SLAB_D // LANES)]
                    for q in range(SLAB_D // LANES):
                        sb[r, pl.ds(q * LANES, LANES)] = a[q] * splat

        pltpu.async_copy(sb, shared.at[mbufs[m][1].at[k]], ssems[sslot],
                         add=True)

        if k < CH - NGB:
            pltpu.async_copy(
                x_in.at[mbufs[m][0].at[k + NGB]], gb, gsems[gslot])
        elif has_next:
            pltpu.async_copy(
                x_in.at[mbufs[mn][0].at[k + NGB - CH]], gb, gsems[gslot])

    def chunk(s, ch, m, has_next, guard_first=False):
        if has_next:
            stage_meta(s, ch + 1, 1 - m)
        for k in range(CH):
            visit(s, ch, k, m, has_next, guard_first)

    @pl.loop(0, 2)
    def _pass(sp):
        s = 2 * c + sp

        @pl.loop(0, BATCH)
        def _zfill(i):
            for q in range(SLAB_D // LANES):
                gb0[i, pl.ds(q * LANES, LANES)] = zero16

        @pl.loop(0, ROWS_PER_TILE // BATCH)
        def _zero(k):
            pltpu.sync_copy(
                gb0, shared.at[pl.ds(w * ROWS_PER_TILE + k * BATCH, BATCH)])
        plsc.subcore_barrier()

        stage_meta(s, 0, 0)
        wait_meta(s, 0, 0)
        for k in range(NGB):
            pltpu.async_copy(x_in.at[mbufs[0][0].at[k]], gbufs[k], gsems[k])

        @pl.loop(0, nc - 1, step=2)
        def _pair(ch):
            chunk(s, ch, 0, True, guard_first=True)
            chunk(s, ch + 1, 1, True)

        chunk(s, nc - 1, 0, False)

        for k in range(NSB):
            pltpu.make_async_copy(
                sbufs[k], shared.at[mbufs[0][1].at[k]], ssems[k]).wait()
        plsc.subcore_barrier()

        @pl.loop(0, ROWS_PER_TILE // BATCH)
        def _wb(k2):
            base = w * ROWS_PER_TILE + k2 * BATCH
            pltpu.sync_copy(shared.at[pl.ds(base, BATCH)],
                            out_hbm.at[pl.ds(s * N + base, BATCH)])
        plsc.subcore_barrier()


def _make_layer(nb):
    mesh = plsc.VectorSubcoreMesh(core_axis_name="c", subcore_axis_name="s")
    meta = [
        pltpu.VMEM((CH, BATCH), jnp.int32),
        pltpu.VMEM((CH, BATCH), jnp.int32),
        pltpu.VMEM((CH, BATCH), jnp.float32),
    ]
    return pl.kernel(
        _layer_body,
        out_type=jax.ShapeDtypeStruct((SLABS * N, SLAB_D), jnp.float32),
        mesh=mesh,
        compiler_params=pltpu.CompilerParams(use_tc_tiling_on_sc=False),
        scratch_types=[
            pltpu.VMEM_SHARED((N, SLAB_D), jnp.float32),
            *meta, *meta,
            pltpu.VMEM((BATCH, SLAB_D), jnp.float32),
            pltpu.VMEM((BATCH, SLAB_D), jnp.float32),
            pltpu.VMEM((BATCH, SLAB_D), jnp.float32),
            pltpu.VMEM((BATCH, SLAB_D), jnp.float32),
            pltpu.VMEM((BATCH, SLAB_D), jnp.float32),
            pltpu.VMEM((BATCH, SLAB_D), jnp.float32),
            pltpu.SemaphoreType.DMA,
            pltpu.SemaphoreType.DMA,
            pltpu.SemaphoreType.DMA,
            pltpu.SemaphoreType.DMA,
            pltpu.SemaphoreType.DMA,
            pltpu.SemaphoreType.DMA,
            pltpu.SemaphoreType.DMA,
            pltpu.SemaphoreType.DMA,
        ],
    )


def kernel(x, rows, cols, vals):
    nnz = rows.shape[0]
    group = NT * BATCH * CH
    nnz_pad = -(-nnz // group) * group
    nb = nnz_pad // (NT * BATCH)
    if (nb // CH) % 2 == 0:
        nnz_pad += group
        nb += CH
    pad = nnz_pad - nnz

    cols_p = jnp.pad(cols, (0, pad))
    rows_p = jnp.pad(rows, (0, pad))
    vals_p = jnp.pad(vals, (0, pad))

    cols_t = cols_p.reshape(NT, nb, BATCH)
    idx4 = cols_t[None, ...] + (jnp.arange(SLABS, dtype=jnp.int32) * N)[
        :, None, None, None]
    rows_t = rows_p.reshape(NT, nb, BATCH)
    vals_t = vals_p.reshape(NT, nb, BATCH)

    xt = x.reshape(N, SLABS, SLAB_D).transpose(1, 0, 2).reshape(
        SLABS * N, SLAB_D)

    layer = _make_layer(nb)
    for _ in range(LAYERS):
        xt = layer(xt, idx4, rows_t, vals_t)

    return xt.reshape(SLABS, N, SLAB_D).transpose(1, 0, 2).reshape(N, D)
